# baseline (speedup 1.0000x reference)
import functools

import jax
import jax.numpy as jnp
from jax import lax
from jax.experimental import pallas as pl
from jax.experimental.pallas import tpu as pltpu
from jax.experimental.pallas import tpu_sc as plsc

_L = 16
_B = 128


def _cubic_weights(t):
    A = -0.75
    t2 = t * t
    t3 = t2 * t
    s0 = t + 1.0
    w0 = A * ((s0 * s0) * s0 - 5.0 * (s0 * s0) + 8.0 * s0 - 4.0)
    w1 = (A + 2.0) * t3 - (A + 3.0) * t2 + 1.0
    s2 = 1.0 - t
    w2 = (A + 2.0) * ((s2 * s2) * s2) - (A + 3.0) * (s2 * s2) + 1.0
    s3 = 2.0 - t
    w3 = A * ((s3 * s3) * s3 - 5.0 * (s3 * s3) + 8.0 * s3 - 4.0)
    return w0, w1, w2, w3


@functools.lru_cache(maxsize=None)
def _build_table_kernel(h, w, nc, ns):
    hp = h + 3
    nw = nc * ns
    base_rows = hp // nw
    rem = hp - base_rows * nw
    mesh = plsc.VectorSubcoreMesh(core_axis_name="c", subcore_axis_name="s")
    unroll = 32
    n_int = (w - 4) // (2 * unroll)

    def body(img_hbm, table_hbm, ch_v, row_v, sem_in, sem_out):
        wid = lax.axis_index("s") * nc + lax.axis_index("c")
        iota = lax.iota(jnp.int32, _L)
        klane = iota & 7
        xoff = iota >> 3
        p_lo = klane * 2
        p_hi = p_lo + 1

        def addr_const(p):
            return (p & 3) * (w + 8) + xoff + (p >> 2) - 1

        clo = addr_const(p_lo)
        chi = addr_const(p_hi)
        nrows_w = base_rows + jnp.where(wid < rem, 1, 0).astype(jnp.int32)

        def start_in(t, slot):
            yp = wid + t * nw
            y = jnp.minimum(jnp.maximum(yp - 1, 0), h - 1)
            for c in range(3):
                pltpu.async_copy(
                    img_hbm.at[pl.ds(c * h * w + y * w, w)],
                    ch_v.at[slot].at[pl.ds(c * (w + 8), w)],
                    sem_in.at[slot],
                )

        def wait_in(t, slot):
            yp = wid + t * nw
            y = jnp.minimum(jnp.maximum(yp - 1, 0), h - 1)
            for c in range(3):
                pltpu.make_async_copy(
                    img_hbm.at[pl.ds(c * h * w + y * w, w)],
                    ch_v.at[slot].at[pl.ds(c * (w + 8), w)],
                    sem_in.at[slot],
                ).wait()

        table_2d = table_hbm

        def out_desc(t, slot, half):
            yp = wid + t * nw
            return pltpu.make_async_copy(
                row_v.at[slot].at[pl.ds(half * (w // 2), w // 2)],
                table_2d.at[pl.ds(yp * w + half * (w // 2), w // 2)],
                sem_out.at[slot],
            )

        def row_helpers(slot):
            cv = ch_v.at[slot]

            def pack2(lo_f, hi_f):
                lo_i = plsc.bitcast(lo_f, jnp.int32)
                hi_i = plsc.bitcast(hi_f, jnp.int32)
                lo_r = lax.shift_right_logical(lo_i + 0x8000, 16)
                hi_r = (hi_i + 0x8000) & jnp.int32(-65536)
                return lo_r | hi_r

            def put_pair(x0s, word):
                plsc.store_scatter(row_v.at[slot], [xoff + x0s, klane], word)

            def edge_pair(x0):
                pxl = jnp.minimum(
                    jnp.maximum(x0 + xoff + (p_lo >> 2) - 1, 0), w - 1
                )
                pxh = jnp.minimum(
                    jnp.maximum(x0 + xoff + (p_hi >> 2) - 1, 0), w - 1
                )
                lo = plsc.load_gather(cv, [(p_lo & 3) * (w + 8) + pxl])
                hi = plsc.load_gather(cv, [(p_hi & 3) * (w + 8) + pxh])
                put_pair(x0, pack2(lo, hi))

            def interior(lo_x0, n_pairs):
                nf = n_pairs // unroll

                def win_body(j, carry2):
                    x0b = j * unroll * 2 + lo_x0
                    for u in range(unroll):
                        x0s = x0b + 2 * u
                        lo = plsc.load_gather(cv, [clo + x0s])
                        hi = plsc.load_gather(cv, [chi + x0s])
                        put_pair(x0s, pack2(lo, hi))
                    return carry2

                lax.fori_loop(0, nf, win_body, 0)
                for x0 in range(
                    lo_x0 + nf * unroll * 2, lo_x0 + n_pairs * 2, 2
                ):
                    lo = plsc.load_gather(cv, [clo + x0])
                    hi = plsc.load_gather(cv, [chi + x0])
                    put_pair(x0, pack2(lo, hi))

            return edge_pair, interior

        start_in(0, 0)

        def pair_body(tt, carry):
            for u in (0, 1):
                t = tt * 2 + u

                @pl.when(t < nrows_w)
                def _():
                    @pl.when(t + 1 < nrows_w)
                    def _():
                        start_in(t + 1, 1 - u)

                    wait_in(t, u)

                    @pl.when(t >= 2)
                    def _():
                        out_desc(t - 2, u, 0).wait()
                        out_desc(t - 2, u, 1).wait()

                    edge_pair, interior = row_helpers(u)
                    n_half = (w // 2 - 2) // 2
                    edge_pair(0)
                    interior(2, n_half)
                    out_desc(t, u, 0).start()
                    interior(w // 2, n_half)
                    edge_pair(w - 2)
                    out_desc(t, u, 1).start()

            return carry

        lax.fori_loop(0, (base_rows + 2) // 2, pair_body, 0)

        for half in (0, 1):
            out_desc(nrows_w - 2, (base_rows + 1) % 2, half).wait()
            out_desc(nrows_w - 1, base_rows % 2, half).wait()

    return pl.kernel(
        body,
        out_type=jax.ShapeDtypeStruct((hp * w, 8), jnp.int32),
        mesh=mesh,
        compiler_params=pltpu.CompilerParams(
            needs_layout_passes=False,
            use_tc_tiling_on_sc=False,
            disable_bounds_checks=True,
        ),
        scratch_types=[
            pltpu.VMEM((2, 4 * (w + 8)), jnp.float32),
            pltpu.VMEM((2, w, 8), jnp.int32),
            pltpu.SemaphoreType.DMA((2,)),
            pltpu.SemaphoreType.DMA((2,)),
        ],
    )


@functools.lru_cache(maxsize=None)
def _build_sc_call(nq, h, w, nc, ns):
    nblk = -(-nq // _B)
    tail_q = nq - (nblk - 1) * _B
    assert tail_q % _L == 0 and nq % _L == 0
    m_rows = (h + 3) * w
    nw = nc * ns
    base_blocks = nblk // nw
    rem = nblk - base_blocks * nw
    tail_wid = (nblk - 1) % nw
    groups = _B // _L

    mesh = plsc.VectorSubcoreMesh(core_axis_name="c", subcore_axis_name="s")

    def body(xs_hbm, table_hbm, out_hbm, xs_v, wbuf, idx_v, rows_v, out_v,
             sem_x, sem_g, sem_o):
        wid = lax.axis_index("s") * nc + lax.axis_index("c")
        iota = lax.iota(jnp.int32, _L)
        nblk_w = base_blocks + jnp.where(wid < rem, 1, 0).astype(jnp.int32)

        def xs_descs(t, s):
            blk = wid + t * nw
            base = blk * _B
            n = jnp.where(blk == (nblk - 1), tail_q, _B)
            full = [
                pltpu.make_async_copy(
                    xs_hbm.at[pl.ds(base, _B)],
                    xs_v.at[s].at[pl.ds(0, _B)],
                    sem_x.at[s],
                ),
                pltpu.make_async_copy(
                    xs_hbm.at[pl.ds(nq + base, _B)],
                    xs_v.at[s].at[pl.ds(_B, _B)],
                    sem_x.at[s],
                ),
            ]
            tail = [
                pltpu.make_async_copy(
                    xs_hbm.at[pl.ds(base, tail_q)],
                    xs_v.at[s].at[pl.ds(0, tail_q)],
                    sem_x.at[s],
                ),
                pltpu.make_async_copy(
                    xs_hbm.at[pl.ds(nq + base, tail_q)],
                    xs_v.at[s].at[pl.ds(_B, tail_q)],
                    sem_x.at[s],
                ),
            ]
            del n
            return full, tail, blk == (nblk - 1)

        def xs_start(t, s):
            full, tail, is_tail = xs_descs(t, s)

            @pl.when(jnp.logical_not(is_tail))
            def _():
                for d in full:
                    d.start()

            @pl.when(is_tail)
            def _():
                for d in tail:
                    d.start()

        def xs_wait(t, s):
            full, tail, is_tail = xs_descs(t, s)

            @pl.when(jnp.logical_not(is_tail))
            def _():
                for d in full:
                    d.wait()

            @pl.when(is_tail)
            def _():
                for d in tail:
                    d.wait()

        def phase_a(t, s):
            for g in range(groups):
                sl = pl.ds(g * _L, _L)
                x = xs_v[s, sl]
                y = xs_v[s, pl.ds(_B + g * _L, _L)]
                gx = 2.0 * x - 1.0
                gy = 2.0 * y - 1.0
                ixf = (gx + 1.0) / 2.0 * float(w - 1)
                iyf = (gy + 1.0) / 2.0 * float(h - 1)
                x0 = ixf.astype(jnp.int32)
                y0 = iyf.astype(jnp.int32)
                x0 = jnp.minimum(jnp.maximum(x0, 0), w - 1)
                y0 = jnp.minimum(jnp.maximum(y0, 0), h - 1)
                tx = ixf - x0.astype(jnp.float32)
                ty = iyf - y0.astype(jnp.float32)
                wx = _cubic_weights(tx)
                wy = _cubic_weights(ty)
                for j in range(4):
                    wbuf[s, j, sl] = wx[j]
                    wbuf[s, 4 + j, sl] = wy[j]
                rbase = y0 * w + x0
                for i in range(4):
                    r = rbase + i * w
                    r = jnp.minimum(jnp.maximum(r, 0), m_rows - 1)
                    idx_v[s, i, sl] = r

        def gather_descs(s):
            return [
                pltpu.make_async_copy(
                    table_hbm.at[idx_v.at[s].at[i]],
                    rows_v.at[s].at[pl.ds(i * _B, _B)],
                    sem_g.at[s],
                )
                for i in range(4)
            ]

        def phase_b(t, s):
            for g in range(groups):
                sl = pl.ds(g * _L, _L)
                q = iota + (g * _L)
                wx = [wbuf[s, j, sl] for j in range(4)]
                wy = [wbuf[s, 4 + j, sl] for j in range(4)]
                acc = [None, None, None]
                for i in range(4):
                    rowv = q + (i * _B)
                    wik = [wy[i] * wx[k] for k in range(4)]
                    for j in range(8):
                        pos = jnp.full((_L,), j, jnp.int32)
                        wv = plsc.load_gather(rows_v.at[s], [rowv, pos])
                        p0 = 2 * j
                        v0 = plsc.bitcast(lax.shift_left(wv, 16), jnp.float32)
                        c0 = p0 & 3
                        contrib = wik[p0 >> 2] * v0
                        if acc[c0] is None:
                            acc[c0] = contrib
                        else:
                            acc[c0] = acc[c0] + contrib
                        p1 = 2 * j + 1
                        c1 = p1 & 3
                        if c1 < 3:
                            v1 = plsc.bitcast(
                                wv & jnp.int32(-65536), jnp.float32
                            )
                            contrib1 = wik[p1 >> 2] * v1
                            if acc[c1] is None:
                                acc[c1] = contrib1
                            else:
                                acc[c1] = acc[c1] + contrib1
                for c in range(3):
                    out_v[s, c, sl] = acc[c]

        def out_full_desc(t, s):
            blk = wid + t * nw
            return pltpu.make_async_copy(
                out_v.at[s],
                out_hbm.at[blk],
                sem_o.at[s],
            )

        out_tail_desc = out_full_desc

        def out_start(t, s):
            blk = wid + t * nw
            is_tail = blk == (nblk - 1)

            @pl.when(jnp.logical_not(is_tail))
            def _():
                out_full_desc(t, s).start()

            @pl.when(is_tail)
            def _():
                out_tail_desc(t, s).start()

        def step(t, s):
            @pl.when(t < nblk_w)
            def _():
                xs_wait(t, s)
                phase_a(t, s)
                for d in gather_descs(s):
                    d.start()

                @pl.when(t + 1 < nblk_w)
                def _():
                    xs_start(t + 1, 1 - s)

            @pl.when(jnp.logical_and(t >= 1, t <= nblk_w))
            def _():
                s1 = 1 - s
                for d in gather_descs(s1):
                    d.wait()

                @pl.when(t >= 3)
                def _():
                    blk3 = wid + (t - 3) * nw
                    is_tail3 = blk3 == (nblk - 1)

                    @pl.when(jnp.logical_not(is_tail3))
                    def _():
                        out_full_desc(t - 3, s1).wait()

                    @pl.when(is_tail3)
                    def _():
                        out_tail_desc(t - 3, s1).wait()

                phase_b(t - 1, s1)
                out_start(t - 1, s1)

        xs_start(0, 0)

        def pair_body(tt, carry):
            step(tt * 2, 0)
            step(tt * 2 + 1, 1)
            return carry

        lax.fori_loop(0, (base_blocks + 1 + 2) // 2, pair_body, 0)

        lastpar = (base_blocks % 2, (base_blocks - 1) % 2)

        @pl.when(wid == tail_wid)
        def _():
            out_tail_desc(base_blocks, lastpar[0]).wait()
            out_full_desc(base_blocks - 1, lastpar[1]).wait()

        @pl.when(jnp.logical_and(wid < rem, wid != tail_wid))
        def _():
            out_full_desc(base_blocks, lastpar[0]).wait()
            out_full_desc(base_blocks - 1, lastpar[1]).wait()

        @pl.when(wid >= rem)
        def _():
            out_full_desc(base_blocks - 1, lastpar[1]).wait()
            out_full_desc(base_blocks - 2, lastpar[0]).wait()

    return pl.kernel(
        body,
        out_type=jax.ShapeDtypeStruct((nblk, 4, _B), jnp.float32),
        mesh=mesh,
        compiler_params=pltpu.CompilerParams(
            needs_layout_passes=False,
            use_tc_tiling_on_sc=False,
            disable_bounds_checks=True,
        ),
        scratch_types=[
            pltpu.VMEM((2, 2 * _B), jnp.float32),
            pltpu.VMEM((2, 8, _B), jnp.float32),
            pltpu.VMEM((2, 4, _B), jnp.int32),
            pltpu.VMEM((2, 4 * _B, 8), jnp.int32),
            pltpu.VMEM((2, 4, _B), jnp.float32),
            pltpu.SemaphoreType.DMA((2,)),
            pltpu.SemaphoreType.DMA((2,)),
            pltpu.SemaphoreType.DMA((2,)),
        ],
    )


def kernel(xs, image):
    c, h, w = image.shape
    nq = xs.shape[0]
    info = plsc.get_sparse_core_info()
    tab_fn = _build_table_kernel(h, w, info.num_cores, info.num_subcores)
    table = tab_fn(image.reshape(c * h * w))
    fn = _build_sc_call(nq, h, w, info.num_cores, info.num_subcores)
    out4 = fn(xs.T.reshape(2 * nq), table)
    nblk = out4.shape[0]
    out = out4[:, :3, :].transpose(0, 2, 1).reshape(nblk * _B, 3)
    return out[:nq]


# speedup vs baseline: 1.0217x; 1.0217x over previous
import functools

import jax
import jax.numpy as jnp
from jax import lax
from jax.experimental import pallas as pl
from jax.experimental.pallas import tpu as pltpu
from jax.experimental.pallas import tpu_sc as plsc

_L = 16
_B = 128


def _cubic_weights(t):
    A = -0.75
    t2 = t * t
    t3 = t2 * t
    s0 = t + 1.0
    w0 = A * ((s0 * s0) * s0 - 5.0 * (s0 * s0) + 8.0 * s0 - 4.0)
    w1 = (A + 2.0) * t3 - (A + 3.0) * t2 + 1.0
    s2 = 1.0 - t
    w2 = (A + 2.0) * ((s2 * s2) * s2) - (A + 3.0) * (s2 * s2) + 1.0
    s3 = 2.0 - t
    w3 = A * ((s3 * s3) * s3 - 5.0 * (s3 * s3) + 8.0 * s3 - 4.0)
    return w0, w1, w2, w3


@functools.lru_cache(maxsize=None)
def _build_table_kernel(h, w, nc, ns):
    hp = h + 3
    nw = nc * ns
    base_rows = hp // nw
    rem = hp - base_rows * nw
    mesh = plsc.VectorSubcoreMesh(core_axis_name="c", subcore_axis_name="s")
    unroll = 32
    n_int = (w - 4) // (2 * unroll)

    def body(img_hbm, table_hbm, ch_v, row_v, sem_in, sem_out):
        wid = lax.axis_index("s") * nc + lax.axis_index("c")
        iota = lax.iota(jnp.int32, _L)
        klane = iota & 7
        xoff = iota >> 3
        p_lo = klane * 2
        p_hi = p_lo + 1

        def addr_const(p):
            return (p & 3) * (w + 8) + xoff + (p >> 2) - 1

        clo = addr_const(p_lo)
        chi = addr_const(p_hi)
        nrows_w = base_rows + jnp.where(wid < rem, 1, 0).astype(jnp.int32)

        def start_in(t, slot):
            yp = wid + t * nw
            y = jnp.minimum(jnp.maximum(yp - 1, 0), h - 1)
            for c in range(3):
                pltpu.async_copy(
                    img_hbm.at[pl.ds(c * h * w + y * w, w)],
                    ch_v.at[slot].at[pl.ds(c * (w + 8), w)],
                    sem_in.at[slot],
                )

        def wait_in(t, slot):
            yp = wid + t * nw
            y = jnp.minimum(jnp.maximum(yp - 1, 0), h - 1)
            for c in range(3):
                pltpu.make_async_copy(
                    img_hbm.at[pl.ds(c * h * w + y * w, w)],
                    ch_v.at[slot].at[pl.ds(c * (w + 8), w)],
                    sem_in.at[slot],
                ).wait()

        table_2d = table_hbm

        def out_desc(t, slot):
            yp = wid + t * nw
            return pltpu.make_async_copy(
                row_v.at[slot],
                table_2d.at[pl.ds(yp * w, w)],
                sem_out.at[slot],
            )

        def compute_row(slot):
            cv = ch_v.at[slot]

            def pack2(lo_f, hi_f):
                lo_i = plsc.bitcast(lo_f, jnp.int32)
                hi_i = plsc.bitcast(hi_f, jnp.int32)
                lo_r = lax.shift_right_logical(lo_i + 0x8000, 16)
                hi_r = (hi_i + 0x8000) & jnp.int32(-65536)
                return lo_r | hi_r

            def put_pair(x0s, word):
                plsc.store_scatter(row_v.at[slot], [xoff + x0s, klane], word)

            for x0 in (0, w - 2):
                pxl = jnp.minimum(
                    jnp.maximum(x0 + xoff + (p_lo >> 2) - 1, 0), w - 1
                )
                pxh = jnp.minimum(
                    jnp.maximum(x0 + xoff + (p_hi >> 2) - 1, 0), w - 1
                )
                lo = plsc.load_gather(cv, [(p_lo & 3) * (w + 8) + pxl])
                hi = plsc.load_gather(cv, [(p_hi & 3) * (w + 8) + pxh])
                put_pair(x0, pack2(lo, hi))

            def win_body(j, carry2):
                x0b = j * unroll * 2 + 2
                for u in range(unroll):
                    x0s = x0b + 2 * u
                    lo = plsc.load_gather(cv, [clo + x0s])
                    hi = plsc.load_gather(cv, [chi + x0s])
                    put_pair(x0s, pack2(lo, hi))
                return carry2

            lax.fori_loop(0, n_int, win_body, 0)
            for x0 in range(2 + n_int * unroll * 2, w - 2, 2):
                lo = plsc.load_gather(cv, [clo + x0])
                hi = plsc.load_gather(cv, [chi + x0])
                put_pair(x0, pack2(lo, hi))

        start_in(0, 0)

        def pair_body(tt, carry):
            for u in (0, 1):
                t = tt * 2 + u

                @pl.when(t < nrows_w)
                def _():
                    @pl.when(t + 1 < nrows_w)
                    def _():
                        start_in(t + 1, 1 - u)

                    wait_in(t, u)

                    @pl.when(t >= 2)
                    def _():
                        out_desc(t - 2, u).wait()

                    compute_row(u)
                    out_desc(t, u).start()

            return carry

        lax.fori_loop(0, (base_rows + 2) // 2, pair_body, 0)

        out_desc(nrows_w - 2, (base_rows + 1) % 2).wait()
        out_desc(nrows_w - 1, base_rows % 2).wait()

    return pl.kernel(
        body,
        out_type=jax.ShapeDtypeStruct((hp * w, 8), jnp.int32),
        mesh=mesh,
        compiler_params=pltpu.CompilerParams(
            needs_layout_passes=False,
            use_tc_tiling_on_sc=False,
            disable_bounds_checks=True,
        ),
        scratch_types=[
            pltpu.VMEM((2, 4 * (w + 8)), jnp.float32),
            pltpu.VMEM((2, w, 8), jnp.int32),
            pltpu.SemaphoreType.DMA((2,)),
            pltpu.SemaphoreType.DMA((2,)),
        ],
    )


@functools.lru_cache(maxsize=None)
def _build_sc_call(nq, h, w, nc, ns):
    nblk = -(-nq // _B)
    tail_q = nq - (nblk - 1) * _B
    assert tail_q % _L == 0 and nq % _L == 0
    m_rows = (h + 3) * w
    nw = nc * ns
    base_blocks = nblk // nw
    rem = nblk - base_blocks * nw
    tail_wid = (nblk - 1) % nw
    groups = _B // _L

    mesh = plsc.VectorSubcoreMesh(core_axis_name="c", subcore_axis_name="s")

    def body(xs_hbm, table_hbm, out_hbm, xs_v, wbuf, idx_v, rows_v, out_v,
             sem_x, sem_g, sem_o):
        wid = lax.axis_index("s") * nc + lax.axis_index("c")
        iota = lax.iota(jnp.int32, _L)
        nblk_w = base_blocks + jnp.where(wid < rem, 1, 0).astype(jnp.int32)

        def xs_descs(t, s):
            blk = wid + t * nw
            base = blk * _B
            n = jnp.where(blk == (nblk - 1), tail_q, _B)
            full = [
                pltpu.make_async_copy(
                    xs_hbm.at[pl.ds(base, _B)],
                    xs_v.at[s].at[pl.ds(0, _B)],
                    sem_x.at[s],
                ),
                pltpu.make_async_copy(
                    xs_hbm.at[pl.ds(nq + base, _B)],
                    xs_v.at[s].at[pl.ds(_B, _B)],
                    sem_x.at[s],
                ),
            ]
            tail = [
                pltpu.make_async_copy(
                    xs_hbm.at[pl.ds(base, tail_q)],
                    xs_v.at[s].at[pl.ds(0, tail_q)],
                    sem_x.at[s],
                ),
                pltpu.make_async_copy(
                    xs_hbm.at[pl.ds(nq + base, tail_q)],
                    xs_v.at[s].at[pl.ds(_B, tail_q)],
                    sem_x.at[s],
                ),
            ]
            del n
            return full, tail, blk == (nblk - 1)

        def xs_start(t, s):
            full, tail, is_tail = xs_descs(t, s)

            @pl.when(jnp.logical_not(is_tail))
            def _():
                for d in full:
                    d.start()

            @pl.when(is_tail)
            def _():
                for d in tail:
                    d.start()

        def xs_wait(t, s):
            full, tail, is_tail = xs_descs(t, s)

            @pl.when(jnp.logical_not(is_tail))
            def _():
                for d in full:
                    d.wait()

            @pl.when(is_tail)
            def _():
                for d in tail:
                    d.wait()

        def phase_a(t, s):
            for g in range(groups):
                sl = pl.ds(g * _L, _L)
                x = xs_v[s, sl]
                y = xs_v[s, pl.ds(_B + g * _L, _L)]
                gx = 2.0 * x - 1.0
                gy = 2.0 * y - 1.0
                ixf = (gx + 1.0) / 2.0 * float(w - 1)
                iyf = (gy + 1.0) / 2.0 * float(h - 1)
                x0 = ixf.astype(jnp.int32)
                y0 = iyf.astype(jnp.int32)
                x0 = jnp.minimum(jnp.maximum(x0, 0), w - 1)
                y0 = jnp.minimum(jnp.maximum(y0, 0), h - 1)
                tx = ixf - x0.astype(jnp.float32)
                ty = iyf - y0.astype(jnp.float32)
                wx = _cubic_weights(tx)
                wy = _cubic_weights(ty)
                for j in range(4):
                    wbuf[s, j, sl] = wx[j]
                    wbuf[s, 4 + j, sl] = wy[j]
                rbase = y0 * w + x0
                for i in range(4):
                    r = rbase + i * w
                    r = jnp.minimum(jnp.maximum(r, 0), m_rows - 1)
                    idx_v[s, i, sl] = r

        def gather_descs(s):
            return [
                pltpu.make_async_copy(
                    table_hbm.at[idx_v.at[s].at[i]],
                    rows_v.at[s].at[pl.ds(i * _B, _B)],
                    sem_g.at[s],
                )
                for i in range(4)
            ]

        def phase_b(t, s):
            for g in range(groups):
                sl = pl.ds(g * _L, _L)
                q = iota + (g * _L)
                wx = [wbuf[s, j, sl] for j in range(4)]
                wy = [wbuf[s, 4 + j, sl] for j in range(4)]
                acc = [None, None, None]
                for i in range(4):
                    rowv = q + (i * _B)
                    wik = [wy[i] * wx[k] for k in range(4)]
                    for j in range(8):
                        pos = jnp.full((_L,), j, jnp.int32)
                        wv = plsc.load_gather(rows_v.at[s], [rowv, pos])
                        p0 = 2 * j
                        v0 = plsc.bitcast(lax.shift_left(wv, 16), jnp.float32)
                        c0 = p0 & 3
                        contrib = wik[p0 >> 2] * v0
                        if acc[c0] is None:
                            acc[c0] = contrib
                        else:
                            acc[c0] = acc[c0] + contrib
                        p1 = 2 * j + 1
                        c1 = p1 & 3
                        if c1 < 3:
                            v1 = plsc.bitcast(
                                wv & jnp.int32(-65536), jnp.float32
                            )
                            contrib1 = wik[p1 >> 2] * v1
                            if acc[c1] is None:
                                acc[c1] = contrib1
                            else:
                                acc[c1] = acc[c1] + contrib1
                for c in range(3):
                    out_v[s, c, sl] = acc[c]

        def out_full_desc(t, s):
            blk = wid + t * nw
            return pltpu.make_async_copy(
                out_v.at[s],
                out_hbm.at[blk],
                sem_o.at[s],
            )

        out_tail_desc = out_full_desc

        def out_start(t, s):
            blk = wid + t * nw
            is_tail = blk == (nblk - 1)

            @pl.when(jnp.logical_not(is_tail))
            def _():
                out_full_desc(t, s).start()

            @pl.when(is_tail)
            def _():
                out_tail_desc(t, s).start()

        def step(t, s):
            @pl.when(t < nblk_w)
            def _():
                xs_wait(t, s)
                phase_a(t, s)
                for d in gather_descs(s):
                    d.start()

                @pl.when(t + 1 < nblk_w)
                def _():
                    xs_start(t + 1, 1 - s)

            @pl.when(jnp.logical_and(t >= 1, t <= nblk_w))
            def _():
                s1 = 1 - s
                for d in gather_descs(s1):
                    d.wait()

                @pl.when(t >= 3)
                def _():
                    blk3 = wid + (t - 3) * nw
                    is_tail3 = blk3 == (nblk - 1)

                    @pl.when(jnp.logical_not(is_tail3))
                    def _():
                        out_full_desc(t - 3, s1).wait()

                    @pl.when(is_tail3)
                    def _():
                        out_tail_desc(t - 3, s1).wait()

                phase_b(t - 1, s1)
                out_start(t - 1, s1)

        xs_start(0, 0)

        def pair_body(tt, carry):
            step(tt * 2, 0)
            step(tt * 2 + 1, 1)
            return carry

        lax.fori_loop(0, (base_blocks + 1 + 2) // 2, pair_body, 0)

        lastpar = (base_blocks % 2, (base_blocks - 1) % 2)

        @pl.when(wid == tail_wid)
        def _():
            out_tail_desc(base_blocks, lastpar[0]).wait()
            out_full_desc(base_blocks - 1, lastpar[1]).wait()

        @pl.when(jnp.logical_and(wid < rem, wid != tail_wid))
        def _():
            out_full_desc(base_blocks, lastpar[0]).wait()
            out_full_desc(base_blocks - 1, lastpar[1]).wait()

        @pl.when(wid >= rem)
        def _():
            out_full_desc(base_blocks - 1, lastpar[1]).wait()
            out_full_desc(base_blocks - 2, lastpar[0]).wait()

    return pl.kernel(
        body,
        out_type=jax.ShapeDtypeStruct((nblk, 4, _B), jnp.float32),
        mesh=mesh,
        compiler_params=pltpu.CompilerParams(
            needs_layout_passes=False,
            use_tc_tiling_on_sc=False,
            disable_bounds_checks=True,
        ),
        scratch_types=[
            pltpu.VMEM((2, 2 * _B), jnp.float32),
            pltpu.VMEM((2, 8, _B), jnp.float32),
            pltpu.VMEM((2, 4, _B), jnp.int32),
            pltpu.VMEM((2, 4 * _B, 8), jnp.int32),
            pltpu.VMEM((2, 4, _B), jnp.float32),
            pltpu.SemaphoreType.DMA((2,)),
            pltpu.SemaphoreType.DMA((2,)),
            pltpu.SemaphoreType.DMA((2,)),
        ],
    )


def kernel(xs, image):
    c, h, w = image.shape
    nq = xs.shape[0]
    info = plsc.get_sparse_core_info()
    tab_fn = _build_table_kernel(h, w, info.num_cores, info.num_subcores)
    table = tab_fn(image.reshape(c * h * w))
    fn = _build_sc_call(nq, h, w, info.num_cores, info.num_subcores)
    out4 = fn(xs.T.reshape(2 * nq), table)
    nblk = out4.shape[0]
    out = out4[:, :3, :].transpose(0, 2, 1).reshape(nblk * _B, 3)
    return out[:nq]
